# baseline (speedup 1.0000x reference)
import jax
import jax.numpy as jnp
from jax.experimental import pallas as pl
from jax.experimental.pallas import tpu as pltpu


def _shift_body(data_ref, x_ref, out_ref, sem0, sem1):
    n = data_ref.shape[0]
    bulk = pltpu.make_async_copy(
        data_ref.at[pl.ds(0, n - 1), :],
        out_ref.at[pl.ds(1, n - 1), :],
        sem0,
    )
    head = pltpu.make_async_copy(
        x_ref,
        out_ref.at[pl.ds(0, 1), :],
        sem1,
    )
    bulk.start()
    head.start()
    head.wait()
    bulk.wait()


def kernel(data, x):
    n, d = data.shape
    return pl.pallas_call(
        _shift_body,
        in_specs=[
            pl.BlockSpec(memory_space=pl.ANY),
            pl.BlockSpec(memory_space=pl.ANY),
        ],
        out_specs=pl.BlockSpec(memory_space=pl.ANY),
        out_shape=jax.ShapeDtypeStruct((n, d), data.dtype),
        scratch_shapes=[pltpu.SemaphoreType.DMA, pltpu.SemaphoreType.DMA],
    )(data, x.reshape(1, d))


# speedup vs baseline: 1.0027x; 1.0027x over previous
import jax
import jax.numpy as jnp
from jax.experimental import pallas as pl
from jax.experimental.pallas import tpu as pltpu

_NCHUNK = 32


def _shift_body(data_ref, x_ref, out_ref, sems, head_sem):
    n = data_ref.shape[0]
    cb = n // _NCHUNK
    copies = []
    for w in range(_NCHUNK):
        src0 = w * cb
        rows = cb if w < _NCHUNK - 1 else cb - 1
        c = pltpu.make_async_copy(
            data_ref.at[pl.ds(src0, rows), :],
            out_ref.at[pl.ds(src0 + 1, rows), :],
            sems.at[w],
        )
        c.start()
        copies.append(c)
    head = pltpu.make_async_copy(x_ref, out_ref.at[pl.ds(0, 1), :], head_sem)
    head.start()
    head.wait()
    for c in copies:
        c.wait()


def kernel(data, x):
    n, d = data.shape
    return pl.pallas_call(
        _shift_body,
        in_specs=[
            pl.BlockSpec(memory_space=pl.ANY),
            pl.BlockSpec(memory_space=pl.ANY),
        ],
        out_specs=pl.BlockSpec(memory_space=pl.ANY),
        out_shape=jax.ShapeDtypeStruct((n, d), data.dtype),
        scratch_shapes=[
            pltpu.SemaphoreType.DMA((_NCHUNK,)),
            pltpu.SemaphoreType.DMA,
        ],
    )(data, x.reshape(1, d))


# speedup vs baseline: 31.0276x; 30.9444x over previous
import jax
import jax.numpy as jnp
from jax.experimental import pallas as pl
from jax.experimental.pallas import tpu as pltpu

_BLOCK = 2048


def _shift_body(a_ref, p_ref, x_ref, out_ref):
    i = pl.program_id(0)
    out_ref[:] = pltpu.roll(a_ref[:], 1, 0)

    @pl.when(i == 0)
    def _():
        out_ref[0:1] = x_ref[:]

    @pl.when(i > 0)
    def _():
        out_ref[0:1] = p_ref[7:8]


def kernel(data, x):
    n, d = data.shape
    nb = n // _BLOCK
    tiles_per_block = _BLOCK // 8
    return pl.pallas_call(
        _shift_body,
        grid=(nb,),
        in_specs=[
            pl.BlockSpec((_BLOCK, d), lambda i: (i, 0)),
            pl.BlockSpec(
                (8, d),
                lambda i: (jnp.maximum(i * tiles_per_block - 1, 0), 0),
            ),
            pl.BlockSpec((1, d), lambda i: (0, 0)),
        ],
        out_specs=pl.BlockSpec((_BLOCK, d), lambda i: (i, 0)),
        out_shape=jax.ShapeDtypeStruct((n, d), data.dtype),
    )(data, data, x.reshape(1, d))
